# Initial kernel scaffold; baseline (speedup 1.0000x reference)
import jax
import jax.numpy as jnp
from jax.experimental import pallas as pl


def kernel(h, b, norm, edge_index):
    raise NotImplementedError("write your pallas kernel here")


# speedup vs baseline: 3.0049x; 3.0049x over previous
import functools

import jax
import jax.numpy as jnp
from jax import lax
from jax.experimental import pallas as pl
from jax.experimental.pallas import tpu as pltpu
from jax.experimental.pallas import tpu_sc as plsc

NC = 2
NS = 16
NW = NC * NS
CHUNK = 128
L = 16


def _prescale_body(h_ref, norm_ref, o_ref):
    o_ref[...] = h_ref[...] * norm_ref[...]


def _finish_body(h_ref, b_ref, norm_ref, a0_ref, a1_ref, d0_ref, d1_ref,
                 hout_ref, bout_ref):
    deg = jnp.maximum(d0_ref[...] + d1_ref[...], 1.0)
    c = (a0_ref[...] + a1_ref[...]) / deg
    bb = b_ref[...]
    l2 = jnp.sqrt(jnp.sum(bb * bb, axis=1, keepdims=True) +
                  jnp.sum(c * c, axis=1, keepdims=True))
    inv = 1.0 / jnp.maximum(l2, 1e-12)
    d = bb.shape[1]
    bout_ref[:, :d] = bb * inv
    bout_ref[:, d:] = c * inv
    hout_ref[...] = h_ref[...] + c * norm_ref[...]


def _make_sc_aggregate(n_pad, d, chunks_per_worker):
    rows_per_tile = n_pad // NS
    mesh = plsc.VectorSubcoreMesh(core_axis_name="c", subcore_axis_name="s",
                                  num_cores=NC, num_subcores=NS)

    @functools.partial(
        pl.kernel,
        out_type=[
            jax.ShapeDtypeStruct((NC, n_pad, d), jnp.float32),
            jax.ShapeDtypeStruct((NC, n_pad, d), jnp.float32),
        ],
        mesh=mesh,
        scratch_types=[
            pltpu.VMEM((CHUNK,), jnp.int32),
            pltpu.VMEM((CHUNK,), jnp.int32),
            pltpu.VMEM((CHUNK, d), jnp.float32),
            pltpu.VMEM_SHARED((n_pad, d), jnp.float32),
            pltpu.SemaphoreType.DMA,
        ],
    )
    def sc_aggregate(hs_hbm, src_hbm, dst_hbm, zero_hbm, ones_hbm,
                     acc_out, deg_out,
                     src_v, dst_v, rows_v, acc_sh, sem):
        cid = lax.axis_index("c")
        sid = lax.axis_index("s")
        wid = cid * NS + sid
        eoff = wid * (chunks_per_worker * CHUNK)

        base = sid * rows_per_tile
        pltpu.sync_copy(zero_hbm, acc_sh.at[pl.ds(base, rows_per_tile)])
        plsc.subcore_barrier()

        def chunk_body(i, carry):
            off = pl.multiple_of(eoff + i * CHUNK, CHUNK)
            pltpu.sync_copy(src_hbm.at[pl.ds(off, CHUNK)], src_v)
            pltpu.sync_copy(dst_hbm.at[pl.ds(off, CHUNK)], dst_v)
            pltpu.async_copy(hs_hbm.at[src_v], rows_v, sem).wait()
            pltpu.sync_copy(rows_v, acc_sh.at[dst_v], add=True)
            return carry

        lax.fori_loop(0, chunks_per_worker, chunk_body, 0)
        plsc.subcore_barrier()

        pltpu.sync_copy(acc_sh.at[pl.ds(base, rows_per_tile)],
                        acc_out.at[cid, pl.ds(base, rows_per_tile)])
        plsc.subcore_barrier()

        pltpu.sync_copy(zero_hbm, acc_sh.at[pl.ds(base, rows_per_tile)])
        pltpu.sync_copy(ones_hbm, rows_v)
        plsc.subcore_barrier()

        def deg_body(i, carry):
            off = pl.multiple_of(eoff + i * CHUNK, CHUNK)
            pltpu.sync_copy(dst_hbm.at[pl.ds(off, CHUNK)], dst_v)
            pltpu.sync_copy(rows_v, acc_sh.at[dst_v], add=True)
            return carry

        lax.fori_loop(0, chunks_per_worker, deg_body, 0)
        plsc.subcore_barrier()
        pltpu.sync_copy(acc_sh.at[pl.ds(base, rows_per_tile)],
                        deg_out.at[cid, pl.ds(base, rows_per_tile)])

    return sc_aggregate


def kernel(h, b, norm, edge_index):
    n, d = h.shape
    e = edge_index.shape[1]

    epw = -(-e // (NW * 8 * CHUNK)) * 8 * CHUNK
    e_pad = epw * NW
    n_pad = -(-(n + 1) // CHUNK) * CHUNK

    src = edge_index[0].astype(jnp.int32)
    dst = edge_index[1].astype(jnp.int32)
    pad = e_pad - e
    src = jnp.concatenate([src, jnp.zeros((pad,), jnp.int32)])
    dst = jnp.concatenate([dst, jnp.full((pad,), n, jnp.int32)])

    zero_blk = jnp.zeros((n_pad // NS, d), jnp.float32)
    ones_blk = jnp.ones((CHUNK, d), jnp.float32)

    rows_blk = 1000
    grid = n // rows_blk
    hs = pl.pallas_call(
        _prescale_body,
        grid=(grid,),
        in_specs=[
            pl.BlockSpec((rows_blk, d), lambda i: (i, 0)),
            pl.BlockSpec((rows_blk, 1), lambda i: (i, 0)),
        ],
        out_specs=pl.BlockSpec((rows_blk, d), lambda i: (i, 0)),
        out_shape=jax.ShapeDtypeStruct((n, d), jnp.float32),
    )(h, norm)

    sc_aggregate = _make_sc_aggregate(n_pad, d, epw // CHUNK)
    acc, deg = sc_aggregate(hs, src, dst, zero_blk, ones_blk)

    h_new, bundle = pl.pallas_call(
        _finish_body,
        grid=(grid,),
        in_specs=[
            pl.BlockSpec((rows_blk, d), lambda i: (i, 0)),
            pl.BlockSpec((rows_blk, d), lambda i: (i, 0)),
            pl.BlockSpec((rows_blk, 1), lambda i: (i, 0)),
            pl.BlockSpec((rows_blk, d), lambda i: (i, 0)),
            pl.BlockSpec((rows_blk, d), lambda i: (i, 0)),
            pl.BlockSpec((rows_blk, 1), lambda i: (i, 0)),
            pl.BlockSpec((rows_blk, 1), lambda i: (i, 0)),
        ],
        out_specs=[
            pl.BlockSpec((rows_blk, d), lambda i: (i, 0)),
            pl.BlockSpec((rows_blk, 2 * d), lambda i: (i, 0)),
        ],
        out_shape=[
            jax.ShapeDtypeStruct((n, d), jnp.float32),
            jax.ShapeDtypeStruct((n, 2 * d), jnp.float32),
        ],
    )(h, b, norm, acc[0, :n], acc[1, :n], deg[0, :n, :1], deg[1, :n, :1])

    return (h_new, bundle)


# speedup vs baseline: 3.3154x; 1.1033x over previous
import functools

import jax
import jax.numpy as jnp
from jax import lax
from jax.experimental import pallas as pl
from jax.experimental.pallas import tpu as pltpu
from jax.experimental.pallas import tpu_sc as plsc

NC = 2
NS = 16
NW = NC * NS
CHUNK = 128
L = 16


def _prescale_body(h_ref, norm_ref, o_ref):
    o_ref[...] = h_ref[...] * norm_ref[...]


def _finish_body(h_ref, b_ref, norm_ref, a0_ref, a1_ref, d0_ref, d1_ref,
                 hout_ref, bout_ref):
    deg = jnp.maximum(d0_ref[...] + d1_ref[...], 1.0)
    c = (a0_ref[...] + a1_ref[...]) / deg
    bb = b_ref[...]
    l2 = jnp.sqrt(jnp.sum(bb * bb, axis=1, keepdims=True) +
                  jnp.sum(c * c, axis=1, keepdims=True))
    inv = 1.0 / jnp.maximum(l2, 1e-12)
    d = bb.shape[1]
    bout_ref[:, :d] = bb * inv
    bout_ref[:, d:] = c * inv
    hout_ref[...] = h_ref[...] + c * norm_ref[...]


def _make_sc_aggregate(n_pad, d, chunks_per_worker):
    rows_per_tile = n_pad // NS
    half_chunks = chunks_per_worker // 2
    half_edges = half_chunks * CHUNK
    mesh = plsc.VectorSubcoreMesh(core_axis_name="c", subcore_axis_name="s",
                                  num_cores=NC, num_subcores=NS)

    @functools.partial(
        pl.kernel,
        out_type=[
            jax.ShapeDtypeStruct((NC, n_pad, d), jnp.float32),
            jax.ShapeDtypeStruct((NC, n_pad, d), jnp.float32),
        ],
        mesh=mesh,
        scratch_types=[
            pltpu.VMEM((half_edges,), jnp.int32),
            pltpu.VMEM((half_edges,), jnp.int32),
            pltpu.VMEM((CHUNK,), jnp.int32),
            pltpu.VMEM((CHUNK, d), jnp.float32),
            pltpu.VMEM((CHUNK, d), jnp.float32),
            pltpu.VMEM_SHARED((n_pad, d), jnp.float32),
            pltpu.SemaphoreType.DMA,
            pltpu.SemaphoreType.DMA,
        ],
    )
    def sc_aggregate(hs_hbm, src_hbm, dst_hbm, zero_hbm, ones_hbm,
                     acc_out, deg_out,
                     src_v, dst_v, dstc_v, rows_a, rows_b, acc_sh,
                     sem_a, sem_b):
        cid = lax.axis_index("c")
        sid = lax.axis_index("s")
        wid = cid * NS + sid
        eoff = wid * (chunks_per_worker * CHUNK)

        base = sid * rows_per_tile
        pltpu.sync_copy(zero_hbm, acc_sh.at[pl.ds(base, rows_per_tile)])
        plsc.subcore_barrier()

        def gather(c, rows, sem):
            idx = src_v.at[pl.ds(c * CHUNK, CHUNK)]
            return pltpu.async_copy(hs_hbm.at[idx], rows, sem)

        def scatter(c, rows):
            for j in range(CHUNK // 16):
                dstc_v[pl.ds(j * 16, 16)] = dst_v[pl.ds(c * CHUNK + j * 16,
                                                        16)]
            pltpu.sync_copy(rows, acc_sh.at[dstc_v], add=True)

        for hh in range(2):
            off = pl.multiple_of(eoff + hh * half_edges, CHUNK)
            pltpu.sync_copy(src_hbm.at[pl.ds(off, half_edges)], src_v)
            pltpu.sync_copy(dst_hbm.at[pl.ds(off, half_edges)], dst_v)
            gather(0, rows_a, sem_a).wait()

            def pair_body(k, carry):
                gather(2 * k + 1, rows_b, sem_b)
                scatter(2 * k, rows_a)
                sem_b_wait = pltpu.make_async_copy(
                    hs_hbm.at[src_v.at[pl.ds(0, CHUNK)]], rows_b, sem_b)
                sem_b_wait.wait()
                gather(2 * k + 2, rows_a, sem_a)
                scatter(2 * k + 1, rows_b)
                pltpu.make_async_copy(
                    hs_hbm.at[src_v.at[pl.ds(0, CHUNK)]], rows_a,
                    sem_a).wait()
                return carry

            lax.fori_loop(0, half_chunks // 2 - 1, pair_body, 0)
            gather(half_chunks - 1, rows_b, sem_b)
            scatter(half_chunks - 2, rows_a)
            pltpu.make_async_copy(
                hs_hbm.at[src_v.at[pl.ds(0, CHUNK)]], rows_b, sem_b).wait()
            scatter(half_chunks - 1, rows_b)

        plsc.subcore_barrier()
        pltpu.sync_copy(acc_sh.at[pl.ds(base, rows_per_tile)],
                        acc_out.at[cid, pl.ds(base, rows_per_tile)])
        plsc.subcore_barrier()

        pltpu.sync_copy(zero_hbm, acc_sh.at[pl.ds(base, rows_per_tile)])
        pltpu.sync_copy(ones_hbm, rows_a)
        plsc.subcore_barrier()

        for hh in range(2):
            off = pl.multiple_of(eoff + hh * half_edges, CHUNK)
            pltpu.sync_copy(dst_hbm.at[pl.ds(off, half_edges)], dst_v)

            def deg_body(c, carry):
                for j in range(CHUNK // 16):
                    dstc_v[pl.ds(j * 16, 16)] = dst_v[pl.ds(
                        c * CHUNK + j * 16, 16)]
                pltpu.sync_copy(rows_a, acc_sh.at[dstc_v], add=True)
                return carry

            lax.fori_loop(0, half_chunks, deg_body, 0)

        plsc.subcore_barrier()
        pltpu.sync_copy(acc_sh.at[pl.ds(base, rows_per_tile)],
                        deg_out.at[cid, pl.ds(base, rows_per_tile)])

    return sc_aggregate


def kernel(h, b, norm, edge_index):
    n, d = h.shape
    e = edge_index.shape[1]

    epw = -(-e // (NW * 8 * CHUNK)) * 8 * CHUNK
    e_pad = epw * NW
    n_pad = -(-(n + 1) // CHUNK) * CHUNK

    src = edge_index[0].astype(jnp.int32)
    dst = edge_index[1].astype(jnp.int32)
    pad = e_pad - e
    src = jnp.concatenate([src, jnp.zeros((pad,), jnp.int32)])
    dst = jnp.concatenate([dst, jnp.full((pad,), n, jnp.int32)])

    zero_blk = jnp.zeros((n_pad // NS, d), jnp.float32)
    ones_blk = jnp.ones((CHUNK, d), jnp.float32)

    rows_blk = 1000
    grid = n // rows_blk
    hs = pl.pallas_call(
        _prescale_body,
        grid=(grid,),
        in_specs=[
            pl.BlockSpec((rows_blk, d), lambda i: (i, 0)),
            pl.BlockSpec((rows_blk, 1), lambda i: (i, 0)),
        ],
        out_specs=pl.BlockSpec((rows_blk, d), lambda i: (i, 0)),
        out_shape=jax.ShapeDtypeStruct((n, d), jnp.float32),
    )(h, norm)

    sc_aggregate = _make_sc_aggregate(n_pad, d, epw // CHUNK)
    acc, deg = sc_aggregate(hs, src, dst, zero_blk, ones_blk)

    h_new, bundle = pl.pallas_call(
        _finish_body,
        grid=(grid,),
        in_specs=[
            pl.BlockSpec((rows_blk, d), lambda i: (i, 0)),
            pl.BlockSpec((rows_blk, d), lambda i: (i, 0)),
            pl.BlockSpec((rows_blk, 1), lambda i: (i, 0)),
            pl.BlockSpec((rows_blk, d), lambda i: (i, 0)),
            pl.BlockSpec((rows_blk, d), lambda i: (i, 0)),
            pl.BlockSpec((rows_blk, 1), lambda i: (i, 0)),
            pl.BlockSpec((rows_blk, 1), lambda i: (i, 0)),
        ],
        out_specs=[
            pl.BlockSpec((rows_blk, d), lambda i: (i, 0)),
            pl.BlockSpec((rows_blk, 2 * d), lambda i: (i, 0)),
        ],
        out_shape=[
            jax.ShapeDtypeStruct((n, d), jnp.float32),
            jax.ShapeDtypeStruct((n, 2 * d), jnp.float32),
        ],
    )(h, b, norm, acc[0, :n], acc[1, :n], deg[0, :n, :1], deg[1, :n, :1])

    return (h_new, bundle)


# speedup vs baseline: 3.5556x; 1.0724x over previous
import functools

import jax
import jax.numpy as jnp
from jax import lax
from jax.experimental import pallas as pl
from jax.experimental.pallas import tpu as pltpu
from jax.experimental.pallas import tpu_sc as plsc

NC = 2
NS = 16
NW = NC * NS
CHUNK = 128
L = 16


def _prescale_body(h_ref, norm_ref, o_ref):
    o_ref[...] = h_ref[...] * norm_ref[...]


def _finish_body(h_ref, b_ref, norm_ref, a0_ref, a1_ref, d0_ref, d1_ref,
                 hout_ref, bout_ref):
    deg = jnp.maximum(d0_ref[...] + d1_ref[...], 1.0)
    c = (a0_ref[...] + a1_ref[...]) / deg
    bb = b_ref[...]
    l2 = jnp.sqrt(jnp.sum(bb * bb, axis=1, keepdims=True) +
                  jnp.sum(c * c, axis=1, keepdims=True))
    inv = 1.0 / jnp.maximum(l2, 1e-12)
    d = bb.shape[1]
    bout_ref[:, :d] = bb * inv
    bout_ref[:, d:] = c * inv
    hout_ref[...] = h_ref[...] + c * norm_ref[...]


def _make_sc_aggregate(n_pad, d, chunks_per_worker):
    rows_per_tile = n_pad // NS
    half_chunks = chunks_per_worker // 2
    half_edges = half_chunks * CHUNK
    mesh = plsc.VectorSubcoreMesh(core_axis_name="c", subcore_axis_name="s",
                                  num_cores=NC, num_subcores=NS)

    @functools.partial(
        pl.kernel,
        out_type=[
            jax.ShapeDtypeStruct((NC, n_pad, d), jnp.float32),
            jax.ShapeDtypeStruct((NC, n_pad, d), jnp.float32),
        ],
        mesh=mesh,
        scratch_types=[
            pltpu.VMEM((half_edges,), jnp.int32),
            pltpu.VMEM((half_edges,), jnp.int32),
            pltpu.VMEM((CHUNK,), jnp.int32),
            pltpu.VMEM((CHUNK,), jnp.int32),
            pltpu.VMEM((CHUNK, d), jnp.float32),
            pltpu.VMEM((CHUNK, d), jnp.float32),
            pltpu.VMEM_SHARED((n_pad, d), jnp.float32),
            pltpu.SemaphoreType.DMA,
            pltpu.SemaphoreType.DMA,
            pltpu.SemaphoreType.DMA,
            pltpu.SemaphoreType.DMA,
        ],
    )
    def sc_aggregate(hs_hbm, src_hbm, dst_hbm, zero_hbm, ones_hbm,
                     acc_out, deg_out,
                     src_v, dst_v, dstc_a, dstc_b, rows_a, rows_b, acc_sh,
                     sem_a, sem_b, sem_sa, sem_sb):
        cid = lax.axis_index("c")
        sid = lax.axis_index("s")
        wid = cid * NS + sid
        eoff = wid * (chunks_per_worker * CHUNK)

        base = sid * rows_per_tile
        pltpu.sync_copy(zero_hbm, acc_sh.at[pl.ds(base, rows_per_tile)])
        plsc.subcore_barrier()

        def gather(c, rows, sem):
            idx = src_v.at[pl.ds(c * CHUNK, CHUNK)]
            return pltpu.async_copy(hs_hbm.at[idx], rows, sem)

        def gwait(rows, sem):
            pltpu.make_async_copy(
                hs_hbm.at[src_v.at[pl.ds(0, CHUNK)]], rows, sem).wait()

        def prep(c, dstc):
            for j in range(CHUNK // 16):
                dstc[pl.ds(j * 16, 16)] = dst_v[pl.ds(c * CHUNK + j * 16,
                                                      16)]

        def scat(rows, dstc, sem):
            return pltpu.async_copy(rows, acc_sh.at[dstc], sem, add=True)

        def swait(rows, dstc, sem):
            pltpu.make_async_copy(rows, acc_sh.at[dstc], sem).wait()

        for hh in range(2):
            off = pl.multiple_of(eoff + hh * half_edges, CHUNK)
            pltpu.sync_copy(src_hbm.at[pl.ds(off, half_edges)], src_v)
            pltpu.sync_copy(dst_hbm.at[pl.ds(off, half_edges)], dst_v)
            gather(0, rows_a, sem_a)
            prep(0, dstc_a)
            gwait(rows_a, sem_a)
            gather(1, rows_b, sem_b)
            scat(rows_a, dstc_a, sem_sa)

            def pair_body(k, carry):
                prep(2 * k + 1, dstc_b)
                gwait(rows_b, sem_b)
                swait(rows_a, dstc_a, sem_sa)
                gather(2 * k + 2, rows_a, sem_a)
                scat(rows_b, dstc_b, sem_sb)
                prep(2 * k + 2, dstc_a)
                gwait(rows_a, sem_a)
                swait(rows_b, dstc_b, sem_sb)
                gather(2 * k + 3, rows_b, sem_b)
                scat(rows_a, dstc_a, sem_sa)
                return carry

            lax.fori_loop(0, half_chunks // 2 - 1, pair_body, 0)
            prep(half_chunks - 1, dstc_b)
            gwait(rows_b, sem_b)
            swait(rows_a, dstc_a, sem_sa)
            scat(rows_b, dstc_b, sem_sb)
            swait(rows_b, dstc_b, sem_sb)

        plsc.subcore_barrier()
        pltpu.sync_copy(acc_sh.at[pl.ds(base, rows_per_tile)],
                        acc_out.at[cid, pl.ds(base, rows_per_tile)])
        plsc.subcore_barrier()

        pltpu.sync_copy(zero_hbm, acc_sh.at[pl.ds(base, rows_per_tile)])
        pltpu.sync_copy(ones_hbm, rows_a)
        plsc.subcore_barrier()

        for hh in range(2):
            off = pl.multiple_of(eoff + hh * half_edges, CHUNK)
            pltpu.sync_copy(dst_hbm.at[pl.ds(off, half_edges)], dst_v)

            prep(0, dstc_a)
            scat(rows_a, dstc_a, sem_sa)

            def deg_body(k, carry):
                prep(2 * k + 1, dstc_b)
                swait(rows_a, dstc_a, sem_sa)
                scat(rows_a, dstc_b, sem_sb)
                prep(2 * k + 2, dstc_a)
                swait(rows_a, dstc_b, sem_sb)
                scat(rows_a, dstc_a, sem_sa)
                return carry

            lax.fori_loop(0, half_chunks // 2 - 1, deg_body, 0)
            prep(half_chunks - 1, dstc_b)
            swait(rows_a, dstc_a, sem_sa)
            scat(rows_a, dstc_b, sem_sb)
            swait(rows_a, dstc_b, sem_sb)

        plsc.subcore_barrier()
        pltpu.sync_copy(acc_sh.at[pl.ds(base, rows_per_tile)],
                        deg_out.at[cid, pl.ds(base, rows_per_tile)])

    return sc_aggregate


def kernel(h, b, norm, edge_index):
    n, d = h.shape
    e = edge_index.shape[1]

    epw = -(-e // (NW * 8 * CHUNK)) * 8 * CHUNK
    e_pad = epw * NW
    n_pad = -(-(n + 1) // CHUNK) * CHUNK

    src = edge_index[0].astype(jnp.int32)
    dst = edge_index[1].astype(jnp.int32)
    pad = e_pad - e
    src = jnp.concatenate([src, jnp.zeros((pad,), jnp.int32)])
    dst = jnp.concatenate([dst, jnp.full((pad,), n, jnp.int32)])

    zero_blk = jnp.zeros((n_pad // NS, d), jnp.float32)
    ones_blk = jnp.ones((CHUNK, d), jnp.float32)

    rows_blk = 1000
    grid = n // rows_blk
    hs = pl.pallas_call(
        _prescale_body,
        grid=(grid,),
        in_specs=[
            pl.BlockSpec((rows_blk, d), lambda i: (i, 0)),
            pl.BlockSpec((rows_blk, 1), lambda i: (i, 0)),
        ],
        out_specs=pl.BlockSpec((rows_blk, d), lambda i: (i, 0)),
        out_shape=jax.ShapeDtypeStruct((n, d), jnp.float32),
    )(h, norm)

    sc_aggregate = _make_sc_aggregate(n_pad, d, epw // CHUNK)
    acc, deg = sc_aggregate(hs, src, dst, zero_blk, ones_blk)

    h_new, bundle = pl.pallas_call(
        _finish_body,
        grid=(grid,),
        in_specs=[
            pl.BlockSpec((rows_blk, d), lambda i: (i, 0)),
            pl.BlockSpec((rows_blk, d), lambda i: (i, 0)),
            pl.BlockSpec((rows_blk, 1), lambda i: (i, 0)),
            pl.BlockSpec((rows_blk, d), lambda i: (i, 0)),
            pl.BlockSpec((rows_blk, d), lambda i: (i, 0)),
            pl.BlockSpec((rows_blk, 1), lambda i: (i, 0)),
            pl.BlockSpec((rows_blk, 1), lambda i: (i, 0)),
        ],
        out_specs=[
            pl.BlockSpec((rows_blk, d), lambda i: (i, 0)),
            pl.BlockSpec((rows_blk, 2 * d), lambda i: (i, 0)),
        ],
        out_shape=[
            jax.ShapeDtypeStruct((n, d), jnp.float32),
            jax.ShapeDtypeStruct((n, 2 * d), jnp.float32),
        ],
    )(h, b, norm, acc[0, :n], acc[1, :n], deg[0, :n, :1], deg[1, :n, :1])

    return (h_new, bundle)
